# baseline (speedup 1.0000x reference)
import jax
import jax.numpy as jnp
from jax import lax
from jax.experimental import pallas as pl
from jax.experimental.pallas import tpu as pltpu
from jax.experimental.pallas import tpu_sc as plsc

N = 10000
NP = 10240
E = 320000
EP = 327680
D = 128
K = 128
ET = EP // 32
CHUNKS = ET // K
PAD_DST = N + 16


def _tc_front_body(nh_ref, wt_ref, wsd_ref, c3_ref, ehr_ref, v_ref,
                   z_ref, sd_ref, t2_ref):
    z = jnp.dot(nh_ref[...], wt_ref[...], preferred_element_type=jnp.float32)
    z_ref[...] = z
    sd_ref[...] = (jnp.dot(z, wsd_ref[...], preferred_element_type=jnp.float32)
                   + c3_ref[...])
    t2_ref[...] = jnp.dot(ehr_ref[...], v_ref[...],
                          preferred_element_type=jnp.float32)


def _tc_front(node_h, w_fc_t, w_sd, c3b, edge_hr, v_blk):
    nb = 1024
    eb = 4096
    return pl.pallas_call(
        _tc_front_body,
        grid=(10,),
        in_specs=[
            pl.BlockSpec((nb, D), lambda i: (i, 0)),
            pl.BlockSpec((D, D), lambda i: (0, 0)),
            pl.BlockSpec((D, 2), lambda i: (0, 0)),
            pl.BlockSpec((1, 2), lambda i: (0, 0)),
            pl.BlockSpec((eb, D), lambda i: (i, 0)),
            pl.BlockSpec((D, 8), lambda i: (0, 0)),
        ],
        out_specs=[
            pl.BlockSpec((nb, D), lambda i: (i, 0)),
            pl.BlockSpec((nb, 2), lambda i: (i, 0)),
            pl.BlockSpec((eb, 8), lambda i: (i, 0)),
        ],
        out_shape=[
            jax.ShapeDtypeStruct((NP, D), jnp.float32),
            jax.ShapeDtypeStruct((NP, 2), jnp.float32),
            jax.ShapeDtypeStruct((EP // 8, 8), jnp.float32),
        ],
    )(node_h, w_fc_t, w_sd, c3b, edge_hr, v_blk)


def _sc_scores_body(s_hbm, d_hbm, t_hbm, src_hbm, dst_hbm, z1_hbm,
                    exp_hbm, den_hbm,
                    s_loc, d_loc, srcb, dstb, tb, exb, den_sh):
    cid = lax.axis_index("c")
    sid = lax.axis_index("s")
    ebase = (sid * 2 + cid) * ET

    pltpu.sync_copy(s_hbm, s_loc)
    pltpu.sync_copy(d_hbm, d_loc)
    pltpu.sync_copy(z1_hbm.at[pl.ds(sid * 640, 640)],
                    den_sh.at[pl.ds(sid * 640, 640)])
    plsc.subcore_barrier()

    def chunk(j, carry):
        base = ebase + j * K
        pltpu.sync_copy(src_hbm.at[pl.ds(base, K)], srcb)
        pltpu.sync_copy(dst_hbm.at[pl.ds(base, K)], dstb)
        pltpu.sync_copy(t_hbm.at[pl.ds(base, K)], tb)
        for g in range(K // 16):
            sl = pl.ds(g * 16, 16)
            s16 = plsc.load_gather(s_loc, [srcb[sl]])
            d16 = plsc.load_gather(d_loc, [dstb[sl]])
            a = s16 + d16 + tb[sl]
            e = jnp.maximum(a, 0.01 * a)
            ex = jnp.exp(e)
            gid = base + g * 16 + lax.iota(jnp.int32, 16)
            exb[sl] = jnp.where(gid < E, ex, 0.0)
        pltpu.sync_copy(exb, den_sh.at[dstb], add=True)
        pltpu.sync_copy(exb, exp_hbm.at[pl.ds(base, K)])
        return carry

    lax.fori_loop(0, CHUNKS, chunk, 0)
    plsc.subcore_barrier()
    pltpu.sync_copy(den_sh.at[pl.ds(sid * 640, 640)],
                    den_hbm.at[cid, pl.ds(sid * 640, 640)])


def _sc_scores(s_p, d_p, t_p, src_p, dst_p, zeros1):
    mesh = plsc.VectorSubcoreMesh(core_axis_name="c", subcore_axis_name="s")
    return pl.kernel(
        _sc_scores_body,
        out_type=[
            jax.ShapeDtypeStruct((EP,), jnp.float32),
            jax.ShapeDtypeStruct((2, NP), jnp.float32),
        ],
        mesh=mesh,
        compiler_params=pltpu.CompilerParams(needs_layout_passes=False),
        scratch_types=[
            pltpu.VMEM((NP,), jnp.float32),
            pltpu.VMEM((NP,), jnp.float32),
            pltpu.VMEM((K,), jnp.int32),
            pltpu.VMEM((K,), jnp.int32),
            pltpu.VMEM((K,), jnp.float32),
            pltpu.VMEM((K,), jnp.float32),
            pltpu.VMEM_SHARED((NP,), jnp.float32),
        ],
    )(s_p, d_p, t_p, src_p, dst_p, zeros1)


def _sc_aggr_body(z_hbm, exp_hbm, src_hbm, dst_hbm, z2_hbm,
                  h_hbm,
                  rows, srcb, dstb, exb, h_sh, sem):
    cid = lax.axis_index("c")
    sid = lax.axis_index("s")
    ebase = (sid * 2 + cid) * ET
    i16 = lambda v: jnp.zeros((16,), jnp.int32) + v

    pltpu.sync_copy(z2_hbm.at[pl.ds(sid * 640, 640), :],
                    h_sh.at[pl.ds(sid * 640, 640), :])
    plsc.subcore_barrier()

    def chunk(j, carry):
        base = ebase + j * K
        pltpu.sync_copy(src_hbm.at[pl.ds(base, K)], srcb)
        pltpu.sync_copy(dst_hbm.at[pl.ds(base, K)], dstb)
        pltpu.sync_copy(exp_hbm.at[pl.ds(base, K)], exb)
        pltpu.async_copy(z_hbm.at[srcb], rows, sem).wait()

        def edge_scale(e4, carry2):
            for v in range(4):
                ei = e4 * 4 + v
                al = plsc.load_gather(exb, [i16(ei)])
                for col in range(D // 16):
                    csl = pl.ds(col * 16, 16)
                    rows[ei, csl] = rows[ei, csl] * al
            return carry2

        lax.fori_loop(0, K // 4, edge_scale, 0)
        pltpu.sync_copy(rows, h_sh.at[dstb], add=True)
        return carry

    lax.fori_loop(0, CHUNKS, chunk, 0)
    plsc.subcore_barrier()
    pltpu.sync_copy(h_sh.at[pl.ds(sid * 640, 640), :],
                    h_hbm.at[cid, pl.ds(sid * 640, 640), :])


def _sc_aggr(z, exp_p, src_p, dst_p, zeros2):
    mesh = plsc.VectorSubcoreMesh(core_axis_name="c", subcore_axis_name="s")
    return pl.kernel(
        _sc_aggr_body,
        out_type=jax.ShapeDtypeStruct((2, NP, D), jnp.float32),
        mesh=mesh,
        compiler_params=pltpu.CompilerParams(needs_layout_passes=False),
        scratch_types=[
            pltpu.VMEM((K, D), jnp.float32),
            pltpu.VMEM((K,), jnp.int32),
            pltpu.VMEM((K,), jnp.int32),
            pltpu.VMEM((K,), jnp.float32),
            pltpu.VMEM_SHARED((NP, D), jnp.float32),
            pltpu.SemaphoreType.DMA,
        ],
    )(z, exp_p, src_p, dst_p, zeros2)


def _tc_blend_body(nh_ref, hp_ref, den_ref, o_ref):
    dt = den_ref[0] + den_ref[1]
    recip = jnp.where(dt > 0, 1.0 / dt, 0.0)
    o_ref[...] = (0.5 * nh_ref[...]
                  + 0.5 * (hp_ref[0] + hp_ref[1]) * recip[:, None])


def _tc_blend(node_h, h_part, den_part):
    nb = 1024
    return pl.pallas_call(
        _tc_blend_body,
        grid=(10,),
        in_specs=[
            pl.BlockSpec((nb, D), lambda i: (i, 0)),
            pl.BlockSpec((2, nb, D), lambda i: (0, i, 0)),
            pl.BlockSpec((2, nb), lambda i: (0, i)),
        ],
        out_specs=pl.BlockSpec((nb, D), lambda i: (i, 0)),
        out_shape=jax.ShapeDtypeStruct((N, D), jnp.float32),
    )(node_h, h_part, den_part)


def kernel(node_h, edge_index, edge_h, W_fc, W_edge, b_edge, W_attn):
    f32 = jnp.float32
    src = edge_index[0].astype(jnp.int32)
    dst = edge_index[1].astype(jnp.int32)

    w1 = W_attn[0, :D]
    w2 = W_attn[0, D:2 * D]
    w3 = W_attn[0, 2 * D:]
    v3 = W_edge.T @ w3
    c3 = jnp.dot(b_edge, w3)
    c3b = jnp.stack([c3, jnp.zeros((), f32)]).reshape(1, 2)

    edge_hr = jnp.pad(edge_h, ((0, EP - E), (0, 0))).reshape(EP // 8, D)
    v_blk = jnp.zeros((D, 8), f32).at[jnp.arange(D), jnp.arange(D) // 16].set(
        jnp.tile(v3, 8))
    w_sd = jnp.stack([w1, w2], axis=1)

    z, sd, t2 = _tc_front(node_h, W_fc.T, w_sd, c3b, edge_hr, v_blk)

    s_p = sd[:, 0]
    d_p = sd[:, 1]
    t_p = t2.reshape(EP)
    src_p = jnp.pad(src, (0, EP - E))
    dst_p = jnp.pad(dst, (0, EP - E), constant_values=PAD_DST)
    zeros1 = jnp.zeros((NP,), f32)
    zeros2 = jnp.zeros((NP, D), f32)

    exp_p, den_p = _sc_scores(s_p, d_p, t_p, src_p, dst_p, zeros1)
    h_p = _sc_aggr(z, exp_p, src_p, dst_p, zeros2)
    return _tc_blend(node_h, h_p, den_p)


# speedup vs baseline: 1.1324x; 1.1324x over previous
import jax
import jax.numpy as jnp
from jax import lax
from jax.experimental import pallas as pl
from jax.experimental.pallas import tpu as pltpu
from jax.experimental.pallas import tpu_sc as plsc

N = 10000
NP = 10240
E = 320000
EP = 327680
D = 128
K = 128
ET = EP // 32
CHUNKS = ET // K
PAD_DST = N + 16


def _tc_front_body(nh_ref, wt_ref, wsd_ref, c3_ref, ehr_ref, v_ref,
                   z_ref, sd_ref, t2_ref):
    z = jnp.dot(nh_ref[...], wt_ref[...], preferred_element_type=jnp.float32)
    z_ref[...] = z
    sd_ref[...] = (jnp.dot(z, wsd_ref[...], preferred_element_type=jnp.float32)
                   + c3_ref[...])
    t2_ref[...] = jnp.dot(ehr_ref[...], v_ref[...],
                          preferred_element_type=jnp.float32)


def _tc_front(node_h, w_fc_t, w_sd, c3b, edge_hr, v_blk):
    nb = 1024
    eb = 4096
    return pl.pallas_call(
        _tc_front_body,
        grid=(10,),
        in_specs=[
            pl.BlockSpec((nb, D), lambda i: (i, 0)),
            pl.BlockSpec((D, D), lambda i: (0, 0)),
            pl.BlockSpec((D, 2), lambda i: (0, 0)),
            pl.BlockSpec((1, 2), lambda i: (0, 0)),
            pl.BlockSpec((eb, D), lambda i: (i, 0)),
            pl.BlockSpec((D, 8), lambda i: (0, 0)),
        ],
        out_specs=[
            pl.BlockSpec((nb, D), lambda i: (i, 0)),
            pl.BlockSpec((nb, 2), lambda i: (i, 0)),
            pl.BlockSpec((eb, 8), lambda i: (i, 0)),
        ],
        out_shape=[
            jax.ShapeDtypeStruct((NP, D), jnp.float32),
            jax.ShapeDtypeStruct((NP, 2), jnp.float32),
            jax.ShapeDtypeStruct((EP // 8, 8), jnp.float32),
        ],
    )(node_h, w_fc_t, w_sd, c3b, edge_hr, v_blk)


def _sc_scores_body(s_hbm, d_hbm, t_hbm, src_hbm, dst_hbm, z1_hbm,
                    exp_hbm, den_hbm,
                    s_loc, d_loc, srcb, dstb, tb, exb, den_sh, sema):
    cid = lax.axis_index("c")
    sid = lax.axis_index("s")
    ebase = (sid * 2 + cid) * ET

    pltpu.sync_copy(s_hbm, s_loc)
    pltpu.sync_copy(d_hbm, d_loc)
    pltpu.sync_copy(z1_hbm.at[pl.ds(sid * 640, 640)],
                    den_sh.at[pl.ds(sid * 640, 640)])
    plsc.subcore_barrier()

    def chunk(j, carry):
        base = ebase + j * K
        pltpu.async_copy(src_hbm.at[pl.ds(base, K)], srcb, sema)
        pltpu.async_copy(dst_hbm.at[pl.ds(base, K)], dstb, sema)
        pltpu.async_copy(t_hbm.at[pl.ds(base, K)], tb, sema)
        pltpu.make_async_copy(src_hbm.at[pl.ds(base, K)], srcb, sema).wait()
        pltpu.make_async_copy(dst_hbm.at[pl.ds(base, K)], dstb, sema).wait()
        pltpu.make_async_copy(t_hbm.at[pl.ds(base, K)], tb, sema).wait()
        for g in range(K // 16):
            sl = pl.ds(g * 16, 16)
            s16 = plsc.load_gather(s_loc, [srcb[sl]])
            d16 = plsc.load_gather(d_loc, [dstb[sl]])
            a = s16 + d16 + tb[sl]
            e = jnp.maximum(a, 0.01 * a)
            ex = jnp.exp(e)
            gid = base + g * 16 + lax.iota(jnp.int32, 16)
            exb[sl] = jnp.where(gid < E, ex, 0.0)
        pltpu.sync_copy(exb, den_sh.at[dstb], add=True)
        pltpu.sync_copy(exb, exp_hbm.at[pl.ds(base, K)])
        return carry

    lax.fori_loop(0, CHUNKS, chunk, 0)
    plsc.subcore_barrier()
    pltpu.sync_copy(den_sh.at[pl.ds(sid * 640, 640)],
                    den_hbm.at[cid, pl.ds(sid * 640, 640)])


def _sc_scores(s_p, d_p, t_p, src_p, dst_p, zeros1):
    mesh = plsc.VectorSubcoreMesh(core_axis_name="c", subcore_axis_name="s")
    return pl.kernel(
        _sc_scores_body,
        out_type=[
            jax.ShapeDtypeStruct((EP,), jnp.float32),
            jax.ShapeDtypeStruct((2, NP), jnp.float32),
        ],
        mesh=mesh,
        compiler_params=pltpu.CompilerParams(needs_layout_passes=False),
        scratch_types=[
            pltpu.VMEM((NP,), jnp.float32),
            pltpu.VMEM((NP,), jnp.float32),
            pltpu.VMEM((K,), jnp.int32),
            pltpu.VMEM((K,), jnp.int32),
            pltpu.VMEM((K,), jnp.float32),
            pltpu.VMEM((K,), jnp.float32),
            pltpu.VMEM_SHARED((NP,), jnp.float32),
            pltpu.SemaphoreType.DMA,
        ],
    )(s_p, d_p, t_p, src_p, dst_p, zeros1)


def _sc_aggr_body(z_hbm, exp_hbm, src_hbm, dst_hbm, z2_hbm,
                  h_hbm,
                  rows, srcb, dstb, exb, h_sh, sem):
    cid = lax.axis_index("c")
    sid = lax.axis_index("s")
    ebase = (sid * 2 + cid) * ET
    i16 = lambda v: jnp.zeros((16,), jnp.int32) + v

    pltpu.sync_copy(z2_hbm.at[pl.ds(sid * 640, 640), :],
                    h_sh.at[pl.ds(sid * 640, 640), :])
    plsc.subcore_barrier()

    def chunk(j, carry):
        base = ebase + j * K
        pltpu.async_copy(src_hbm.at[pl.ds(base, K)], srcb, sem)
        pltpu.async_copy(dst_hbm.at[pl.ds(base, K)], dstb, sem)
        pltpu.async_copy(exp_hbm.at[pl.ds(base, K)], exb, sem)
        pltpu.make_async_copy(src_hbm.at[pl.ds(base, K)], srcb, sem).wait()
        pltpu.make_async_copy(dst_hbm.at[pl.ds(base, K)], dstb, sem).wait()
        pltpu.make_async_copy(exp_hbm.at[pl.ds(base, K)], exb, sem).wait()
        pltpu.async_copy(z_hbm.at[srcb], rows, sem).wait()

        def edge_scale(e4, carry2):
            for v in range(4):
                ei = e4 * 4 + v
                al = plsc.load_gather(exb, [i16(ei)])
                for col in range(D // 16):
                    csl = pl.ds(col * 16, 16)
                    rows[ei, csl] = rows[ei, csl] * al
            return carry2

        lax.fori_loop(0, K // 4, edge_scale, 0)
        pltpu.sync_copy(rows, h_sh.at[dstb], add=True)
        return carry

    lax.fori_loop(0, CHUNKS, chunk, 0)
    plsc.subcore_barrier()
    pltpu.sync_copy(h_sh.at[pl.ds(sid * 640, 640), :],
                    h_hbm.at[cid, pl.ds(sid * 640, 640), :])


def _sc_aggr(z, exp_p, src_p, dst_p, zeros2):
    mesh = plsc.VectorSubcoreMesh(core_axis_name="c", subcore_axis_name="s")
    return pl.kernel(
        _sc_aggr_body,
        out_type=jax.ShapeDtypeStruct((2, NP, D), jnp.float32),
        mesh=mesh,
        compiler_params=pltpu.CompilerParams(needs_layout_passes=False),
        scratch_types=[
            pltpu.VMEM((K, D), jnp.float32),
            pltpu.VMEM((K,), jnp.int32),
            pltpu.VMEM((K,), jnp.int32),
            pltpu.VMEM((K,), jnp.float32),
            pltpu.VMEM_SHARED((NP, D), jnp.float32),
            pltpu.SemaphoreType.DMA,
        ],
    )(z, exp_p, src_p, dst_p, zeros2)


def _tc_blend_body(nh_ref, hp_ref, den_ref, o_ref):
    dt = den_ref[0] + den_ref[1]
    recip = jnp.where(dt > 0, 1.0 / dt, 0.0)
    o_ref[...] = (0.5 * nh_ref[...]
                  + 0.5 * (hp_ref[0] + hp_ref[1]) * recip[:, None])


def _tc_blend(node_h, h_part, den_part):
    nb = 1024
    return pl.pallas_call(
        _tc_blend_body,
        grid=(10,),
        in_specs=[
            pl.BlockSpec((nb, D), lambda i: (i, 0)),
            pl.BlockSpec((2, nb, D), lambda i: (0, i, 0)),
            pl.BlockSpec((2, nb), lambda i: (0, i)),
        ],
        out_specs=pl.BlockSpec((nb, D), lambda i: (i, 0)),
        out_shape=jax.ShapeDtypeStruct((N, D), jnp.float32),
    )(node_h, h_part, den_part)


def kernel(node_h, edge_index, edge_h, W_fc, W_edge, b_edge, W_attn):
    f32 = jnp.float32
    src = edge_index[0].astype(jnp.int32)
    dst = edge_index[1].astype(jnp.int32)

    w1 = W_attn[0, :D]
    w2 = W_attn[0, D:2 * D]
    w3 = W_attn[0, 2 * D:]
    v3 = W_edge.T @ w3
    c3 = jnp.dot(b_edge, w3)
    c3b = jnp.stack([c3, jnp.zeros((), f32)]).reshape(1, 2)

    edge_hr = jnp.pad(edge_h, ((0, EP - E), (0, 0))).reshape(EP // 8, D)
    v_blk = jnp.zeros((D, 8), f32).at[jnp.arange(D), jnp.arange(D) // 16].set(
        jnp.tile(v3, 8))
    w_sd = jnp.stack([w1, w2], axis=1)

    z, sd, t2 = _tc_front(node_h, W_fc.T, w_sd, c3b, edge_hr, v_blk)

    s_p = sd[:, 0]
    d_p = sd[:, 1]
    t_p = t2.reshape(EP)
    src_p = jnp.pad(src, (0, EP - E))
    dst_p = jnp.pad(dst, (0, EP - E), constant_values=PAD_DST)
    zeros1 = jnp.zeros((NP,), f32)
    zeros2 = jnp.zeros((NP, D), f32)

    exp_p, den_p = _sc_scores(s_p, d_p, t_p, src_p, dst_p, zeros1)
    h_p = _sc_aggr(z, exp_p, src_p, dst_p, zeros2)
    return _tc_blend(node_h, h_p, den_p)


# speedup vs baseline: 1.4455x; 1.2765x over previous
import jax
import jax.numpy as jnp
from jax import lax
from jax.experimental import pallas as pl
from jax.experimental.pallas import tpu as pltpu
from jax.experimental.pallas import tpu_sc as plsc

N = 10000
NP = 10240
E = 320000
EP = 327680
D = 128
K = 128
ET = EP // 32
CHUNKS = ET // K
PAD_DST = N + 16


def _tc_front_body(nh_ref, wt_ref, wsd_ref, c3_ref, ehr_ref, v_ref,
                   z_ref, sd_ref, t2_ref):
    z = jnp.dot(nh_ref[...], wt_ref[...], preferred_element_type=jnp.float32)
    z_ref[...] = z
    sd_ref[...] = (jnp.dot(z, wsd_ref[...], preferred_element_type=jnp.float32)
                   + c3_ref[...])
    t2_ref[...] = jnp.dot(ehr_ref[...], v_ref[...],
                          preferred_element_type=jnp.float32)


def _tc_front(node_h, w_fc_t, w_sd, c3b, edge_hr, v_blk):
    nb = 1024
    eb = 4096
    return pl.pallas_call(
        _tc_front_body,
        grid=(10,),
        in_specs=[
            pl.BlockSpec((nb, D), lambda i: (i, 0)),
            pl.BlockSpec((D, D), lambda i: (0, 0)),
            pl.BlockSpec((D, 2), lambda i: (0, 0)),
            pl.BlockSpec((1, 2), lambda i: (0, 0)),
            pl.BlockSpec((eb, D), lambda i: (i, 0)),
            pl.BlockSpec((D, 8), lambda i: (0, 0)),
        ],
        out_specs=[
            pl.BlockSpec((nb, D), lambda i: (i, 0)),
            pl.BlockSpec((nb, 2), lambda i: (i, 0)),
            pl.BlockSpec((eb, 8), lambda i: (i, 0)),
        ],
        out_shape=[
            jax.ShapeDtypeStruct((NP, D), jnp.float32),
            jax.ShapeDtypeStruct((NP, 2), jnp.float32),
            jax.ShapeDtypeStruct((EP // 8, 8), jnp.float32),
        ],
    )(node_h, w_fc_t, w_sd, c3b, edge_hr, v_blk)


def _sc_scores_body(s_hbm, d_hbm, t_hbm, src_hbm, dst_hbm, z1_hbm,
                    exp_hbm, den_hbm,
                    s_loc, d_loc, src_a, dst_a, t_a, eb0, eb1, db0, db1,
                    den_sh, sx0, sx1, sw0, sw1):
    cid = lax.axis_index("c")
    sid = lax.axis_index("s")
    cb0 = (sid * 2 + cid) * CHUNKS
    exb = (eb0, eb1)
    dstb = (db0, db1)
    sexp = (sx0, sx1)
    swr = (sw0, sw1)

    pltpu.sync_copy(s_hbm, s_loc)
    pltpu.sync_copy(d_hbm, d_loc)
    pltpu.sync_copy(src_hbm.at[pl.ds(cb0, CHUNKS), :], src_a)
    pltpu.sync_copy(dst_hbm.at[pl.ds(cb0, CHUNKS), :], dst_a)
    pltpu.sync_copy(t_hbm.at[pl.ds(cb0, CHUNKS), :], t_a)
    pltpu.sync_copy(z1_hbm.at[pl.ds(sid * 640, 640)],
                    den_sh.at[pl.ds(sid * 640, 640)])
    plsc.subcore_barrier()

    def process(c, u):
        base = (cb0 + c) * K
        @pl.when(c >= 2)
        def _():
            pltpu.make_async_copy(exb[u], den_sh.at[dstb[u]], sexp[u]).wait()
            pltpu.make_async_copy(exb[u], exp_hbm.at[pl.ds(base, K)],
                                  swr[u]).wait()
        for g in range(K // 16):
            sl = pl.ds(g * 16, 16)
            dst16 = dst_a[c, sl]
            s16 = plsc.load_gather(s_loc, [src_a[c, sl]])
            d16 = plsc.load_gather(d_loc, [dst16])
            a = s16 + d16 + t_a[c, sl]
            e = jnp.maximum(a, 0.01 * a)
            ex = jnp.exp(e)
            gid = base + g * 16 + lax.iota(jnp.int32, 16)
            exb[u][sl] = jnp.where(gid < E, ex, 0.0)
            dstb[u][sl] = dst16
        pltpu.async_copy(exb[u], den_sh.at[dstb[u]], sexp[u], add=True)
        pltpu.async_copy(exb[u], exp_hbm.at[pl.ds(base, K)], swr[u])

    def macro(j, carry):
        process(j * 2, 0)
        process(j * 2 + 1, 1)
        return carry

    lax.fori_loop(0, CHUNKS // 2, macro, 0)
    for c in (CHUNKS - 2, CHUNKS - 1):
        u = c % 2
        base = (cb0 + c) * K
        pltpu.make_async_copy(exb[u], den_sh.at[dstb[u]], sexp[u]).wait()
        pltpu.make_async_copy(exb[u], exp_hbm.at[pl.ds(base, K)],
                              swr[u]).wait()
    plsc.subcore_barrier()
    pltpu.sync_copy(den_sh.at[pl.ds(sid * 640, 640)],
                    den_hbm.at[cid, pl.ds(sid * 640, 640)])


def _sc_scores(s_p, d_p, t2d, src2d, dst2d, zeros1):
    mesh = plsc.VectorSubcoreMesh(core_axis_name="c", subcore_axis_name="s")
    return pl.kernel(
        _sc_scores_body,
        out_type=[
            jax.ShapeDtypeStruct((EP,), jnp.float32),
            jax.ShapeDtypeStruct((2, NP), jnp.float32),
        ],
        mesh=mesh,
        compiler_params=pltpu.CompilerParams(needs_layout_passes=False),
        scratch_types=[
            pltpu.VMEM((NP,), jnp.float32),
            pltpu.VMEM((NP,), jnp.float32),
            pltpu.VMEM((CHUNKS, K), jnp.int32),
            pltpu.VMEM((CHUNKS, K), jnp.int32),
            pltpu.VMEM((CHUNKS, K), jnp.float32),
            pltpu.VMEM((K,), jnp.float32),
            pltpu.VMEM((K,), jnp.float32),
            pltpu.VMEM((K,), jnp.int32),
            pltpu.VMEM((K,), jnp.int32),
            pltpu.VMEM_SHARED((NP,), jnp.float32),
            pltpu.SemaphoreType.DMA,
            pltpu.SemaphoreType.DMA,
            pltpu.SemaphoreType.DMA,
            pltpu.SemaphoreType.DMA,
        ],
    )(s_p, d_p, t2d, src2d, dst2d, zeros1)


def _sc_aggr_body(z_hbm, exp_hbm, src_hbm, dst_hbm, z2_hbm,
                  h_hbm,
                  src_a, d0, d1, e0, e1, g0, g1, h_sh,
                  si0, si1, sw0, sw1, sc0, sc1):
    cid = lax.axis_index("c")
    sid = lax.axis_index("s")
    cb0 = (sid * 2 + cid) * CHUNKS
    dds, exs, gb = (d0, d1), (e0, e1), (g0, g1)
    sidx, srow, ssca = (si0, si1), (sw0, sw1), (sc0, sc1)
    i16 = lambda v: jnp.zeros((16,), jnp.int32) + v

    pltpu.sync_copy(src_hbm.at[pl.ds(cb0, CHUNKS), :], src_a)
    pltpu.sync_copy(z2_hbm.at[pl.ds(sid * 640, 640), :],
                    h_sh.at[pl.ds(sid * 640, 640), :])
    plsc.subcore_barrier()

    pltpu.async_copy(dst_hbm.at[cb0], d0, si0)
    pltpu.async_copy(exp_hbm.at[cb0], e0, si0)
    pltpu.async_copy(z_hbm.at[src_a.at[0]], g0, sw0)

    def process(c, u):
        u1 = 1 - u

        @pl.when(c + 1 < CHUNKS)
        def _():
            @pl.when(c >= 1)
            def _():
                pltpu.make_async_copy(gb[u1], h_sh.at[dds[u1]],
                                      ssca[u1]).wait()
            pltpu.async_copy(dst_hbm.at[cb0 + c + 1], dds[u1], sidx[u1])
            pltpu.async_copy(exp_hbm.at[cb0 + c + 1], exs[u1], sidx[u1])
            pltpu.async_copy(z_hbm.at[src_a.at[c + 1]], gb[u1], srow[u1])

        pltpu.make_async_copy(z_hbm.at[src_a.at[c]], gb[u], srow[u]).wait()
        pltpu.make_async_copy(dst_hbm.at[cb0 + c], dds[u], sidx[u]).wait()
        pltpu.make_async_copy(exp_hbm.at[cb0 + c], exs[u], sidx[u]).wait()

        def edge_scale(e4, carry2):
            for v in range(4):
                ei = e4 * 4 + v
                al = plsc.load_gather(exs[u], [i16(ei)])
                for col in range(D // 16):
                    csl = pl.ds(col * 16, 16)
                    gb[u][ei, csl] = gb[u][ei, csl] * al
            return carry2

        lax.fori_loop(0, K // 4, edge_scale, 0)
        pltpu.async_copy(gb[u], h_sh.at[dds[u]], ssca[u], add=True)

    def macro(j, carry):
        process(j * 2, 0)
        process(j * 2 + 1, 1)
        return carry

    lax.fori_loop(0, CHUNKS // 2, macro, 0)
    for u in range(2):
        pltpu.make_async_copy(gb[u], h_sh.at[dds[u]], ssca[u]).wait()
    plsc.subcore_barrier()
    pltpu.sync_copy(h_sh.at[pl.ds(sid * 640, 640), :],
                    h_hbm.at[cid, pl.ds(sid * 640, 640), :])


def _sc_aggr(z, exp2d, src2d, dst2d, zeros2):
    mesh = plsc.VectorSubcoreMesh(core_axis_name="c", subcore_axis_name="s")
    return pl.kernel(
        _sc_aggr_body,
        out_type=jax.ShapeDtypeStruct((2, NP, D), jnp.float32),
        mesh=mesh,
        compiler_params=pltpu.CompilerParams(needs_layout_passes=False),
        scratch_types=(
            [
                pltpu.VMEM((CHUNKS, K), jnp.int32),
                pltpu.VMEM((K,), jnp.int32),
                pltpu.VMEM((K,), jnp.int32),
                pltpu.VMEM((K,), jnp.float32),
                pltpu.VMEM((K,), jnp.float32),
                pltpu.VMEM((K, D), jnp.float32),
                pltpu.VMEM((K, D), jnp.float32),
                pltpu.VMEM_SHARED((NP, D), jnp.float32),
            ]
            + [pltpu.SemaphoreType.DMA] * 6
        ),
    )(z, exp2d, src2d, dst2d, zeros2)


def _tc_blend_body(nh_ref, hp_ref, den_ref, o_ref):
    dt = den_ref[0] + den_ref[1]
    recip = jnp.where(dt > 0, 1.0 / dt, 0.0)
    o_ref[...] = (0.5 * nh_ref[...]
                  + 0.5 * (hp_ref[0] + hp_ref[1]) * recip[:, None])


def _tc_blend(node_h, h_part, den_part):
    nb = 1024
    return pl.pallas_call(
        _tc_blend_body,
        grid=(10,),
        in_specs=[
            pl.BlockSpec((nb, D), lambda i: (i, 0)),
            pl.BlockSpec((2, nb, D), lambda i: (0, i, 0)),
            pl.BlockSpec((2, nb), lambda i: (0, i)),
        ],
        out_specs=pl.BlockSpec((nb, D), lambda i: (i, 0)),
        out_shape=jax.ShapeDtypeStruct((N, D), jnp.float32),
    )(node_h, h_part, den_part)


def kernel(node_h, edge_index, edge_h, W_fc, W_edge, b_edge, W_attn):
    f32 = jnp.float32
    src = edge_index[0].astype(jnp.int32)
    dst = edge_index[1].astype(jnp.int32)

    w1 = W_attn[0, :D]
    w2 = W_attn[0, D:2 * D]
    w3 = W_attn[0, 2 * D:]
    v3 = W_edge.T @ w3
    c3 = jnp.dot(b_edge, w3)
    c3b = jnp.stack([c3, jnp.zeros((), f32)]).reshape(1, 2)

    edge_hr = jnp.pad(edge_h, ((0, EP - E), (0, 0))).reshape(EP // 8, D)
    v_blk = jnp.zeros((D, 8), f32).at[jnp.arange(D), jnp.arange(D) // 16].set(
        jnp.tile(v3, 8))
    w_sd = jnp.stack([w1, w2], axis=1)

    z, sd, t2 = _tc_front(node_h, W_fc.T, w_sd, c3b, edge_hr, v_blk)

    s_p = sd[:, 0]
    d_p = sd[:, 1]
    t_p = t2.reshape(EP)
    src_p = jnp.pad(src, (0, EP - E))
    dst_p = jnp.pad(dst, (0, EP - E), constant_values=PAD_DST)
    zeros1 = jnp.zeros((NP,), f32)
    zeros2 = jnp.zeros((NP, D), f32)

    src2d = src_p.reshape(EP // K, K)
    dst2d = dst_p.reshape(EP // K, K)
    exp_p, den_p = _sc_scores(s_p, d_p, t_p.reshape(EP // K, K), src2d,
                              dst2d, zeros1)
    h_p = _sc_aggr(z, exp_p.reshape(EP // K, K), src2d, dst2d, zeros2)
    return _tc_blend(node_h, h_p, den_p)
